# initial kernel scaffold (unmeasured)
import jax
import jax.numpy as jnp
from jax import lax
from jax.experimental import pallas as pl
from jax.experimental.pallas import tpu as pltpu

N_DEV = 32
M_PER = 128
K = 4096
N_PER = 256
M = 4096
DEPTH = 4


def _gelu(y):
    c = 0.7978845608028654
    return 0.5 * y * (1.0 + jnp.tanh(c * (y + 0.044715 * y * y * y)))


def kernel(x, w_mat):
    def body(perm_ref, x_ref, w_ref, out_ref,
             x_bf, y_buf, recv_buf, send_sems, recv_sems):
        t = pl.program_id(0)
        i = lax.axis_index("i")
        j = perm_ref[t]

        @pl.when(t == 0)
        def _():
            x_bf[...] = x_ref[...].astype(jnp.bfloat16)

        y = jnp.dot(x_bf[...], w_ref[...].astype(jnp.bfloat16),
                    preferred_element_type=jnp.float32)
        y = _gelu(y).astype(jnp.bfloat16)

        def send_desc(step, slot, target):
            return pltpu.make_async_remote_copy(
                src_ref=y_buf.at[slot],
                dst_ref=recv_buf.at[pl.ds(i * M_PER, M_PER), :],
                send_sem=send_sems.at[step],
                recv_sem=recv_sems.at[i],
                device_id=(target,),
                device_id_type=pl.DeviceIdType.MESH,
            )

        @pl.when(t == 0)
        def _():
            recv_buf[pl.ds(i * M_PER, M_PER), :] = y

        @pl.when(t > 0)
        def _():
            slot = lax.rem(t, DEPTH)

            @pl.when(t >= DEPTH + 1)
            def _():
                prev = t - DEPTH
                send_desc(prev, lax.rem(prev, DEPTH), perm_ref[prev]).wait_send()

            y_buf[slot] = y
            send_desc(t, slot, j).start()

        @pl.when(t == N_DEV - 1)
        def _():
            for step in range(N_DEV - DEPTH, N_DEV):
                send_desc(step, step % DEPTH, perm_ref[step]).wait_send()
            for s in range(N_DEV):
                @pl.when(s != i)
                def _():
                    pltpu.make_async_remote_copy(
                        src_ref=y_buf.at[0],
                        dst_ref=recv_buf.at[pl.ds(s * M_PER, M_PER), :],
                        send_sem=send_sems.at[0],
                        recv_sem=recv_sems.at[s],
                        device_id=(i,),
                        device_id_type=pl.DeviceIdType.MESH,
                    ).wait_recv()
            out_ref[...] = recv_buf[...].astype(jnp.float32)

    i = lax.axis_index("i")
    perm = jnp.remainder(i + jnp.arange(N_DEV, dtype=jnp.int32), N_DEV)

    grid_spec = pltpu.PrefetchScalarGridSpec(
        num_scalar_prefetch=1,
        grid=(N_DEV,),
        in_specs=[
            pl.BlockSpec((M_PER, K), lambda t, p: (0, 0)),
            pl.BlockSpec((K, N_PER), lambda t, p: (0, p[t])),
        ],
        out_specs=pl.BlockSpec((M, N_PER), lambda t, p: (0, 0)),
        scratch_shapes=[
            pltpu.VMEM((M_PER, K), jnp.bfloat16),
            pltpu.VMEM((DEPTH, M_PER, N_PER), jnp.bfloat16),
            pltpu.VMEM((M, N_PER), jnp.bfloat16),
            pltpu.SemaphoreType.DMA((N_DEV,)),
            pltpu.SemaphoreType.DMA((N_DEV,)),
        ],
    )

    return pl.pallas_call(
        body,
        grid_spec=grid_spec,
        out_shape=jax.ShapeDtypeStruct((M, N_PER), jnp.float32),
        compiler_params=pltpu.CompilerParams(
            dimension_semantics=("arbitrary",),
            collective_id=0,
        ),
    )(perm, x, w_mat)


# baseline (device time: 65591 ns/iter reference)
import jax
import jax.numpy as jnp
from jax import lax
from jax.experimental import pallas as pl
from jax.experimental.pallas import tpu as pltpu

N_DEV = 32
M_PER = 128
K = 4096
N_PER = 256
M = 4096
DEPTH = 4


def _gelu(y):
    c = 0.7978845608028654
    return 0.5 * y * (1.0 + jnp.tanh(c * (y + 0.044715 * y * y * y)))


def kernel(x, w_mat):
    def body(perm_ref, x_ref, w_ref, out_ref,
             x_bf, y_buf, recv_buf, send_sems, recv_sems):
        t = pl.program_id(0)
        i = lax.axis_index("i")
        j = perm_ref[t]

        @pl.when(t == 0)
        def _():
            x_bf[...] = x_ref[...].astype(jnp.bfloat16)

        y = jnp.dot(x_bf[...], w_ref[...].astype(jnp.bfloat16),
                    preferred_element_type=jnp.float32)
        y = _gelu(y).astype(jnp.bfloat16)

        def send_desc(step, slot, target):
            return pltpu.make_async_remote_copy(
                src_ref=y_buf.at[slot],
                dst_ref=recv_buf.at[pl.ds(i * M_PER, M_PER), :],
                send_sem=send_sems.at[step],
                recv_sem=recv_sems.at[i],
                device_id=(target,),
                device_id_type=pl.DeviceIdType.MESH,
            )

        @pl.when(t == 0)
        def _():
            recv_buf[pl.ds(i * M_PER, M_PER), :] = y

        @pl.when(t > 0)
        def _():
            slot = lax.rem(t, DEPTH)

            @pl.when(t >= DEPTH + 1)
            def _():
                prev = t - DEPTH
                send_desc(prev, lax.rem(prev, DEPTH), perm_ref[prev]).wait_send()

            y_buf[slot] = y
            send_desc(t, slot, j).start()

        @pl.when(t == N_DEV - 1)
        def _():
            for step in range(N_DEV - DEPTH, N_DEV):
                send_desc(step, step % DEPTH, perm_ref[step]).wait_send()
            for s in range(N_DEV):
                @pl.when(s != i)
                def _():
                    pltpu.make_async_remote_copy(
                        src_ref=y_buf.at[0],
                        dst_ref=recv_buf.at[pl.ds(s * M_PER, M_PER), :],
                        send_sem=send_sems.at[0],
                        recv_sem=recv_sems.at[s],
                        device_id=(i,),
                        device_id_type=pl.DeviceIdType.MESH,
                    ).wait_recv()
            out_ref[...] = recv_buf[...].astype(jnp.float32)

    i = lax.axis_index("i")
    perm = jnp.remainder(i + jnp.arange(N_DEV, dtype=jnp.int32), N_DEV)

    grid_spec = pltpu.PrefetchScalarGridSpec(
        num_scalar_prefetch=1,
        grid=(N_DEV,),
        in_specs=[
            pl.BlockSpec((M_PER, K), lambda t, p: (0, 0)),
            pl.BlockSpec((K, N_PER), lambda t, p: (0, p[t])),
        ],
        out_specs=pl.BlockSpec((M, N_PER), lambda t, p: (0, 0)),
        scratch_shapes=[
            pltpu.VMEM((M_PER, K), jnp.bfloat16),
            pltpu.VMEM((DEPTH, M_PER, N_PER), jnp.bfloat16),
            pltpu.VMEM((M, N_PER), jnp.bfloat16),
            pltpu.SemaphoreType.DMA((N_DEV,)),
            pltpu.SemaphoreType.DMA((N_DEV,)),
        ],
    )

    return pl.pallas_call(
        body,
        grid_spec=grid_spec,
        out_shape=jax.ShapeDtypeStruct((M, N_PER), jnp.float32),
        compiler_params=pltpu.CompilerParams(
            dimension_semantics=("arbitrary",),
        ),
    )(perm, x, w_mat)


# device time: 59519 ns/iter; 1.1020x vs baseline; 1.1020x over previous
import os

import jax
import jax.numpy as jnp
from jax import lax
from jax.experimental import pallas as pl
from jax.experimental.pallas import tpu as pltpu

N_DEV = 32
M_PER = 128
K = 4096
N_PER = 256
M = 4096
DEPTH = 4

_NO_COMM = os.environ.get("KVAR_NO_COMM") == "1"
_NO_COMPUTE = os.environ.get("KVAR_NO_COMPUTE") == "1"


def _gelu(y):
    c = 0.7978845608028654
    return 0.5 * y * (1.0 + jnp.tanh(c * (y + 0.044715 * y * y * y)))


def kernel(x, w_mat):
    def body(perm_ref, x_ref, w_ref, out_ref,
             x_bf, y_buf, recv_buf, send_sems, recv_sems):
        t = pl.program_id(0)
        i = lax.axis_index("i")
        j = perm_ref[t]

        @pl.when(t == 0)
        def _():
            x_bf[...] = x_ref[...].astype(jnp.bfloat16)

        if _NO_COMPUTE:
            y = w_ref[0:M_PER, :].astype(jnp.bfloat16)
        else:
            y = jnp.dot(x_bf[...], w_ref[...].astype(jnp.bfloat16),
                        preferred_element_type=jnp.float32)
            y = _gelu(y).astype(jnp.bfloat16)

        def send_desc(step, slot, target):
            return pltpu.make_async_remote_copy(
                src_ref=y_buf.at[slot],
                dst_ref=recv_buf.at[pl.ds(i * M_PER, M_PER), :],
                send_sem=send_sems.at[step],
                recv_sem=recv_sems.at[i],
                device_id=(target,),
                device_id_type=pl.DeviceIdType.MESH,
            )

        @pl.when(t == 0)
        def _():
            recv_buf[pl.ds(i * M_PER, M_PER), :] = y

        if not _NO_COMM:
            @pl.when(t > 0)
            def _():
                slot = lax.rem(t, DEPTH)

                @pl.when(t >= DEPTH + 1)
                def _():
                    prev = t - DEPTH
                    send_desc(prev, lax.rem(prev, DEPTH),
                              perm_ref[prev]).wait_send()

                y_buf[slot] = y
                send_desc(t, slot, j).start()

        @pl.when(t == N_DEV - 1)
        def _():
            if not _NO_COMM:
                for step in range(N_DEV - DEPTH, N_DEV):
                    send_desc(step, step % DEPTH, perm_ref[step]).wait_send()
                for s in range(N_DEV):
                    @pl.when(s != i)
                    def _():
                        pltpu.make_async_remote_copy(
                            src_ref=y_buf.at[0],
                            dst_ref=recv_buf.at[pl.ds(s * M_PER, M_PER), :],
                            send_sem=send_sems.at[0],
                            recv_sem=recv_sems.at[s],
                            device_id=(i,),
                            device_id_type=pl.DeviceIdType.MESH,
                        ).wait_recv()
            out_ref[...] = recv_buf[...].astype(jnp.float32)

    i = lax.axis_index("i")
    perm = jnp.remainder(i + jnp.arange(N_DEV, dtype=jnp.int32), N_DEV)

    grid_spec = pltpu.PrefetchScalarGridSpec(
        num_scalar_prefetch=1,
        grid=(N_DEV,),
        in_specs=[
            pl.BlockSpec((M_PER, K), lambda t, p: (0, 0)),
            pl.BlockSpec((K, N_PER), lambda t, p: (0, p[t])),
        ],
        out_specs=pl.BlockSpec((M, N_PER), lambda t, p: (0, 0)),
        scratch_shapes=[
            pltpu.VMEM((M_PER, K), jnp.bfloat16),
            pltpu.VMEM((DEPTH, M_PER, N_PER), jnp.bfloat16),
            pltpu.VMEM((M, N_PER), jnp.bfloat16),
            pltpu.SemaphoreType.DMA((N_DEV,)),
            pltpu.SemaphoreType.DMA((N_DEV,)),
        ],
    )

    return pl.pallas_call(
        body,
        grid_spec=grid_spec,
        out_shape=jax.ShapeDtypeStruct((M, N_PER), jnp.float32),
        compiler_params=pltpu.CompilerParams(
            dimension_semantics=("arbitrary",),
        ),
    )(perm, x, w_mat)


# device time: 54084 ns/iter; 1.2128x vs baseline; 1.1005x over previous
import os

import jax
import jax.numpy as jnp
from jax import lax
from jax.experimental import pallas as pl
from jax.experimental.pallas import tpu as pltpu

N_DEV = 32
M_PER = 128
K = 4096
N_PER = 256
M = 4096
DEPTH = 8
LAG = 4

_NO_COMM = os.environ.get("KVAR_NO_COMM") == "1"
_NO_COMPUTE = os.environ.get("KVAR_NO_COMPUTE") == "1"


def _gelu(y):
    c = 0.7978845608028654
    return 0.5 * y * (1.0 + jnp.tanh(c * (y + 0.044715 * y * y * y)))


def kernel(x, w_mat):
    def body(perm_ref, x_ref, w_hbm, out_ref,
             wv, y_buf, recv_buf, fetch_sems, send_sems, recv_sems):
        t = pl.program_id(0)
        i = lax.axis_index("i")
        j = perm_ref[t]

        def fetch_desc(step):
            col = perm_ref[step]
            return pltpu.make_async_copy(
                w_hbm.at[:, pl.ds(col * N_PER, N_PER)],
                wv.at[lax.rem(step, 2)],
                fetch_sems.at[lax.rem(step, 2)],
            )

        @pl.when(t == 0)
        def _():
            fetch_desc(0).start()

        @pl.when(t < N_DEV - 1)
        def _():
            fetch_desc(t + 1).start()

        fetch_desc(t).wait()

        if _NO_COMPUTE:
            y = wv[lax.rem(t, 2), 0:M_PER, :].astype(jnp.bfloat16)
        else:
            y = lax.dot_general(
                x_ref[...], wv[lax.rem(t, 2)],
                (((1,), (0,)), ((), ())),
                precision=lax.Precision.DEFAULT,
                preferred_element_type=jnp.float32,
            )
            y = _gelu(y).astype(jnp.bfloat16)

        def send_desc(step, slot, target):
            return pltpu.make_async_remote_copy(
                src_ref=y_buf.at[slot],
                dst_ref=recv_buf.at[pl.ds(i * M_PER, M_PER), :],
                send_sem=send_sems.at[step],
                recv_sem=recv_sems.at[i],
                device_id=(target,),
                device_id_type=pl.DeviceIdType.MESH,
            )

        def recv_desc(src):
            return pltpu.make_async_remote_copy(
                src_ref=y_buf.at[0],
                dst_ref=recv_buf.at[pl.ds(src * M_PER, M_PER), :],
                send_sem=send_sems.at[0],
                recv_sem=recv_sems.at[src],
                device_id=(i,),
                device_id_type=pl.DeviceIdType.MESH,
            )

        @pl.when(t == 0)
        def _():
            recv_buf[pl.ds(i * M_PER, M_PER), :] = y

        if not _NO_COMM:
            @pl.when(t > 0)
            def _():
                slot = lax.rem(t, DEPTH)

                @pl.when(t >= DEPTH + 1)
                def _():
                    prev = t - DEPTH
                    send_desc(prev, lax.rem(prev, DEPTH),
                              perm_ref[prev]).wait_send()

                y_buf[slot] = y
                send_desc(t, slot, j).start()

            @pl.when(t >= LAG)
            def _():
                c = t - LAG
                s = lax.rem(i - c + N_DEV, N_DEV)

                @pl.when(c > 0)
                def _():
                    recv_desc(s).wait_recv()
                out_ref[pl.ds(s * M_PER, M_PER), :] = (
                    recv_buf[pl.ds(s * M_PER, M_PER), :].astype(jnp.float32))

        @pl.when(t == N_DEV - 1)
        def _():
            if not _NO_COMM:
                for step in range(N_DEV - DEPTH, N_DEV):
                    send_desc(step, step % DEPTH, perm_ref[step]).wait_send()
                for c in range(N_DEV - LAG, N_DEV):
                    s = lax.rem(i - c + N_DEV, N_DEV)
                    recv_desc(s).wait_recv()
                    out_ref[pl.ds(s * M_PER, M_PER), :] = (
                        recv_buf[pl.ds(s * M_PER, M_PER), :]
                        .astype(jnp.float32))
            else:
                out_ref[...] = recv_buf[...].astype(jnp.float32)

    i = lax.axis_index("i")
    perm = jnp.remainder(i + jnp.arange(N_DEV, dtype=jnp.int32), N_DEV)

    grid_spec = pltpu.PrefetchScalarGridSpec(
        num_scalar_prefetch=1,
        grid=(N_DEV,),
        in_specs=[
            pl.BlockSpec((M_PER, K), lambda t, p: (0, 0)),
            pl.BlockSpec(memory_space=pl.ANY),
        ],
        out_specs=pl.BlockSpec((M, N_PER), lambda t, p: (0, 0)),
        scratch_shapes=[
            pltpu.VMEM((2, K, N_PER), jnp.float32),
            pltpu.VMEM((DEPTH, M_PER, N_PER), jnp.bfloat16),
            pltpu.VMEM((M, N_PER), jnp.bfloat16),
            pltpu.SemaphoreType.DMA((2,)),
            pltpu.SemaphoreType.DMA((N_DEV,)),
            pltpu.SemaphoreType.DMA((N_DEV,)),
        ],
    )

    return pl.pallas_call(
        body,
        grid_spec=grid_spec,
        out_shape=jax.ShapeDtypeStruct((M, N_PER), jnp.float32),
        compiler_params=pltpu.CompilerParams(
            dimension_semantics=("arbitrary",),
        ),
    )(perm, x, w_mat)
